# baseline (device time: 9421 ns/iter reference)
import jax
import jax.numpy as jnp
from jax import lax
from jax.experimental import pallas as pl
from jax.experimental.pallas import tpu as pltpu

N_DEV = 4


def kernel(x):
    _, m, n = x.shape

    def body(x_ref, out_ref, send_ref, recv_ref, ssems, rsems):
        my = lax.axis_index("i")
        peers = [
            jnp.bitwise_xor(my, 1),
            3 - my,
            (my + 2) % N_DEV,
        ]

        barrier_sem = pltpu.get_barrier_semaphore()
        for p in peers:
            pl.semaphore_signal(
                barrier_sem, inc=1,
                device_id=(p,), device_id_type=pl.DeviceIdType.MESH,
            )
        pl.semaphore_wait(barrier_sem, 3)

        send_ref[:, :] = x_ref[0].astype(jnp.bfloat16)

        sends = []
        for k, p in enumerate(peers):
            slot = (my - p) % N_DEV
            rdma = pltpu.make_async_remote_copy(
                src_ref=send_ref,
                dst_ref=recv_ref.at[slot],
                send_sem=ssems.at[k],
                recv_sem=rsems.at[slot],
                device_id=(p,),
                device_id_type=pl.DeviceIdType.MESH,
            )
            rdma.start()
            sends.append(rdma)

        recvs = []
        for r in range(1, N_DEV):
            rdma = pltpu.make_async_remote_copy(
                src_ref=send_ref,
                dst_ref=recv_ref.at[r],
                send_sem=ssems.at[0],
                recv_sem=rsems.at[r],
                device_id=(peers[0],),
                device_id_type=pl.DeviceIdType.MESH,
            )
            rdma.wait_recv()
            recvs.append(rdma)

        out_ref[:, :] = (
            (send_ref[:, :] + recv_ref[1]) + (recv_ref[2] + recv_ref[3])
        )

        for rdma in sends:
            rdma.wait_send()

    return pl.pallas_call(
        body,
        out_shape=jax.ShapeDtypeStruct((m, n), jnp.bfloat16),
        in_specs=[pl.BlockSpec(memory_space=pltpu.VMEM)],
        out_specs=pl.BlockSpec(memory_space=pltpu.VMEM),
        scratch_shapes=[
            pltpu.VMEM((m, n), jnp.bfloat16),
            pltpu.VMEM((N_DEV, m, n), jnp.bfloat16),
            pltpu.SemaphoreType.DMA((3,)),
            pltpu.SemaphoreType.DMA((N_DEV,)),
        ],
        compiler_params=pltpu.CompilerParams(collective_id=0),
    )(x)


# device time: 9103 ns/iter; 1.0349x vs baseline; 1.0349x over previous
import jax
import jax.numpy as jnp
from jax import lax
from jax.experimental import pallas as pl
from jax.experimental.pallas import tpu as pltpu


def kernel(x):
    _, m, n = x.shape
    m2 = m // 2

    def body(x_ref, out_ref, xv, res, send_a, recv_a, send_b, recv_b,
             copy_sems, ssems, rsems):
        my = lax.axis_index("i")
        p0 = jnp.bitwise_xor(my, 1)
        p1 = 3 - my

        in_copy = pltpu.make_async_copy(x_ref, xv, copy_sems.at[0])
        in_copy.start()

        barrier_sem = pltpu.get_barrier_semaphore()
        for p in (p0, p1):
            pl.semaphore_signal(
                barrier_sem, inc=1,
                device_id=(p,), device_id_type=pl.DeviceIdType.MESH,
            )
        in_copy.wait()
        xb = xv[0].astype(jnp.bfloat16)
        send_a[0] = xb[:m2]
        send_b[0] = xb[m2:]
        pl.semaphore_wait(barrier_sem, 2)

        rdma_a0 = pltpu.make_async_remote_copy(
            src_ref=send_a.at[0], dst_ref=recv_a.at[0],
            send_sem=ssems.at[0, 0], recv_sem=rsems.at[0, 0],
            device_id=(p0,), device_id_type=pl.DeviceIdType.MESH,
        )
        rdma_b0 = pltpu.make_async_remote_copy(
            src_ref=send_b.at[0], dst_ref=recv_b.at[0],
            send_sem=ssems.at[1, 0], recv_sem=rsems.at[1, 0],
            device_id=(p1,), device_id_type=pl.DeviceIdType.MESH,
        )
        rdma_a0.start()
        rdma_b0.start()

        rdma_a1 = pltpu.make_async_remote_copy(
            src_ref=send_a.at[1], dst_ref=recv_a.at[1],
            send_sem=ssems.at[0, 1], recv_sem=rsems.at[0, 1],
            device_id=(p1,), device_id_type=pl.DeviceIdType.MESH,
        )
        rdma_b1 = pltpu.make_async_remote_copy(
            src_ref=send_b.at[1], dst_ref=recv_b.at[1],
            send_sem=ssems.at[1, 1], recv_sem=rsems.at[1, 1],
            device_id=(p0,), device_id_type=pl.DeviceIdType.MESH,
        )
        rdma_a0.wait_recv()
        rdma_b0.wait_recv()
        send_a[1] = send_a[0] + recv_a[0]
        send_b[1] = send_b[0] + recv_b[0]
        rdma_a1.start()
        rdma_b1.start()

        rdma_a1.wait_recv()
        rdma_b1.wait_recv()
        res[:m2, :] = send_a[1] + recv_a[1]
        res[m2:, :] = send_b[1] + recv_b[1]

        out_copy = pltpu.make_async_copy(res, out_ref, copy_sems.at[1])
        out_copy.start()
        out_copy.wait()

        rdma_a0.wait_send()
        rdma_b0.wait_send()
        rdma_a1.wait_send()
        rdma_b1.wait_send()

    return pl.pallas_call(
        body,
        out_shape=jax.ShapeDtypeStruct((m, n), jnp.bfloat16),
        in_specs=[pl.BlockSpec(memory_space=pl.ANY)],
        out_specs=pl.BlockSpec(memory_space=pl.ANY),
        scratch_shapes=[
            pltpu.VMEM((1, m, n), jnp.float32),
            pltpu.VMEM((m, n), jnp.bfloat16),
            pltpu.VMEM((2, m2, n), jnp.bfloat16),
            pltpu.VMEM((2, m2, n), jnp.bfloat16),
            pltpu.VMEM((2, m2, n), jnp.bfloat16),
            pltpu.VMEM((2, m2, n), jnp.bfloat16),
            pltpu.SemaphoreType.DMA((2,)),
            pltpu.SemaphoreType.DMA((2, 2)),
            pltpu.SemaphoreType.DMA((2, 2)),
        ],
        compiler_params=pltpu.CompilerParams(collective_id=0),
    )(x)
